# baseline (speedup 1.0000x reference)
import functools

import jax
import jax.numpy as jnp
from jax import lax
from jax.experimental import pallas as pl
from jax.experimental.pallas import tpu as pltpu
from jax.experimental.pallas import tpu_sc as plsc

N_NODES = 10000
DIM = 128
N_LAYER = 3
N_GRAPHS = 64
NRBF = 16

NC = 2
NS = 16
NW = NC * NS
C_EDGE = 64
NCH0 = 188
NCH1 = 132
TOTCH = NS * (NCH0 + NCH1)
E_PAD = TOTCH * C_EDGE
EPT = E_PAD // NW
NPAD = 10240
RPT = NPAD // NS

_DOT32 = dict(preferred_element_type=jnp.float32)


_IDX_LO = jnp.asarray([w + 16 * (w // 16) for w in range(DIM // 2)])
_IDX_HI = _IDX_LO + 16


def _bf16_bits(x):
    u = lax.bitcast_convert_type(x, jnp.int32)
    return lax.shift_right_logical(u + 0x8000, 16)


def _dot_nt(a, b):
    return lax.dot_general(a, b, (((1,), (1,)), ((), ())), **_DOT32)


def _embed_body(xr_ref, emb_ref, w0_ref, x_ref, y_ref):
    ids = xr_ref[...]
    cls = lax.broadcasted_iota(jnp.int32, (ids.shape[0], emb_ref.shape[0]), 1)
    onehot = (ids == cls).astype(jnp.float32)
    x = jnp.dot(onehot, emb_ref[...], **_DOT32)
    x_ref[...] = x
    y_ref[...] = jnp.dot(x, w0_ref[...], **_DOT32)


def _embed(x_raw, emb, w_msg0):
    blk = 1000
    grid = N_NODES // blk
    return pl.pallas_call(
        _embed_body,
        grid=(grid,),
        in_specs=[
            pl.BlockSpec((blk, 1), lambda i: (i, 0)),
            pl.BlockSpec(emb.shape, lambda i: (0, 0)),
            pl.BlockSpec((DIM, DIM), lambda i: (0, 0)),
        ],
        out_specs=[
            pl.BlockSpec((blk, DIM), lambda i: (i, 0)),
            pl.BlockSpec((blk, DIM), lambda i: (i, 0)),
        ],
        out_shape=[
            jax.ShapeDtypeStruct((N_NODES, DIM), jnp.float32),
            jax.ShapeDtypeStruct((N_NODES, DIM), jnp.float32),
        ],
    )(x_raw.reshape(N_NODES, 1).astype(jnp.int32), emb, w_msg0)


def _dist2_body(px_h, py_h, pz_h, src_h, dst_h, out_h,
                px, py, pz, sv, dv, ov):
    wid = lax.axis_index("c") * NS + lax.axis_index("s")
    base = wid * EPT
    pltpu.sync_copy(px_h, px)
    pltpu.sync_copy(py_h, py)
    pltpu.sync_copy(pz_h, pz)
    pltpu.sync_copy(src_h.at[pl.ds(base, EPT)], sv)
    pltpu.sync_copy(dst_h.at[pl.ds(base, EPT)], dv)

    def body(k, carry):
        off = k * 16
        si = sv[pl.ds(off, 16)]
        di = dv[pl.ds(off, 16)]
        dx = plsc.load_gather(px, [di]) - plsc.load_gather(px, [si])
        dy = plsc.load_gather(py, [di]) - plsc.load_gather(py, [si])
        dz = plsc.load_gather(pz, [di]) - plsc.load_gather(pz, [si])
        s = dx * dx + dy * dy + dz * dz + 1e-12
        u = plsc.bitcast(s, jnp.int32)
        x = plsc.bitcast(
            lax.shift_right_arithmetic(u, 1) + 0x1FBD1DF5, jnp.float32)
        for _ in range(3):
            x = 0.5 * (x + s / x)
        ov[pl.ds(off, 16)] = x
        return carry

    lax.fori_loop(0, EPT // 16, body, 0)
    pltpu.sync_copy(ov, out_h.at[pl.ds(base, EPT)])


def _edge_dist2(pos_pad, src_flat, dst_flat):
    mesh = plsc.VectorSubcoreMesh(core_axis_name="c", subcore_axis_name="s")
    f = functools.partial(
        pl.kernel,
        out_type=jax.ShapeDtypeStruct((E_PAD,), jnp.float32),
        mesh=mesh,
        compiler_params=pltpu.CompilerParams(needs_layout_passes=False),
        scratch_types=[
            pltpu.VMEM((NPAD,), jnp.float32),
            pltpu.VMEM((NPAD,), jnp.float32),
            pltpu.VMEM((NPAD,), jnp.float32),
            pltpu.VMEM((EPT,), jnp.int32),
            pltpu.VMEM((EPT,), jnp.int32),
            pltpu.VMEM((EPT,), jnp.float32),
        ],
    )(_dist2_body)
    return f(pos_pad[:, 0], pos_pad[:, 1], pos_pad[:, 2], src_flat, dst_flat)


def _attr_body(d_ref, a_ref, wlo_ref, blo_ref, whi_ref, bhi_ref, out_ref):
    d = d_ref[...]
    g = jnp.concatenate([d * d, d, jnp.ones_like(d)], axis=1)
    rbf = jnp.exp(jnp.dot(g, a_ref[...], **_DOT32))
    ulo = lax.bitcast_convert_type(
        jnp.dot(rbf, wlo_ref[...], **_DOT32) + blo_ref[...], jnp.int32)
    uhi = lax.bitcast_convert_type(
        jnp.dot(rbf, whi_ref[...], **_DOT32) + bhi_ref[...], jnp.int32)
    out_ref[...] = lax.bitwise_or(
        lax.bitwise_and(uhi + 0x8000, -65536),
        lax.shift_right_logical(ulo + 0x8000, 16))


def _edge_attr(dist, centers, w_rbf, b_rbf):
    blk = 4096
    grid = E_PAD // blk
    amat = jnp.stack([-jnp.ones((NRBF,)), 2.0 * centers,
                      -centers * centers]).astype(jnp.float32)
    return pl.pallas_call(
        _attr_body,
        grid=(grid,),
        in_specs=[
            pl.BlockSpec((blk, 1), lambda i: (i, 0)),
            pl.BlockSpec((3, NRBF), lambda i: (0, 0)),
            pl.BlockSpec((NRBF, DIM // 2), lambda i: (0, 0)),
            pl.BlockSpec((1, DIM // 2), lambda i: (0, 0)),
            pl.BlockSpec((NRBF, DIM // 2), lambda i: (0, 0)),
            pl.BlockSpec((1, DIM // 2), lambda i: (0, 0)),
        ],
        out_specs=pl.BlockSpec((blk, DIM // 2), lambda i: (i, 0)),
        out_shape=jax.ShapeDtypeStruct((E_PAD, DIM // 2), jnp.int32),
    )(dist.reshape(E_PAD, 1), amat,
      w_rbf[:, _IDX_LO], b_rbf[_IDX_LO].reshape(1, DIM // 2),
      w_rbf[:, _IDX_HI], b_rbf[_IDX_HI].reshape(1, DIM // 2))


def _scatter_body(y_h, attr_h, srcm_h, dstm_h, out_h,
                  ib, r0, r1, a0, a1, m0, shared,
                  si0, si1, si2, si3, sg0, sg1, sa0, sa1):
    c = lax.axis_index("c")
    s = lax.axis_index("s")
    base_ch = jnp.where(c == 0, s * NCH0, NS * NCH0 + s * NCH1)
    my_nch = jnp.where(c == 0, NCH0, NCH1)
    rbufs = (r0, r1)
    abufs = (a0, a1)
    mbufs = (m0, m0)
    sgs = (sg0, sg1)
    sas = (sa0, sa1)
    sis = (si0, si1, si2, si3)

    zv = jnp.zeros((16,), jnp.float32)

    def zrow(r, carry):
        for cc in range(DIM // 16):
            r0[r, pl.ds(cc * 16, 16)] = zv
        return carry

    lax.fori_loop(0, C_EDGE, zrow, 0)
    for r5 in range(RPT // C_EDGE):
        pltpu.sync_copy(r0, shared.at[pl.ds(s * RPT + r5 * C_EDGE, C_EDGE)])
    plsc.subcore_barrier()

    def start_idx(j, q):
        g = base_ch + j
        pltpu.make_async_copy(srcm_h.at[g], ib.at[q, 0], sis[q]).start()
        pltpu.make_async_copy(dstm_h.at[g], ib.at[q, 1], sis[q]).start()

    def start_main(j, q, b):
        g = base_ch + j
        pltpu.make_async_copy(srcm_h.at[g], ib.at[q, 0], sis[q]).wait()
        pltpu.make_async_copy(dstm_h.at[g], ib.at[q, 1], sis[q]).wait()
        pltpu.make_async_copy(y_h.at[ib.at[q, 0]], rbufs[b], sgs[b]).start()
        pltpu.make_async_copy(
            attr_h.at[pl.ds((base_ch + j) * C_EDGE, C_EDGE)],
            abufs[b], sas[b]).start()

    def compute(j, q, b):
        rb, ab, mb = rbufs[b], abufs[b], mbufs[b]
        pltpu.make_async_copy(y_h.at[ib.at[q, 0]], rb, sgs[b]).wait()
        pltpu.make_async_copy(
            attr_h.at[pl.ds((base_ch + j) * C_EDGE, C_EDGE)], ab,
            sas[b]).wait()

        hmask = jnp.full((16,), -65536, jnp.int32)

        def mrow(r4, carry):
            for rr in range(4):
                r = 4 * r4 + rr
                for cc in range(DIM // 32):
                    a32 = ab[r, pl.ds(cc * 16, 16)]
                    lo = plsc.bitcast(lax.shift_left(a32, 16), jnp.float32)
                    hi = plsc.bitcast(lax.bitwise_and(a32, hmask), jnp.float32)
                    sl0 = pl.ds(cc * 32, 16)
                    sl1 = pl.ds(cc * 32 + 16, 16)
                    mb[r, sl0] = rb[r, sl0] * lo
                    mb[r, sl1] = rb[r, sl1] * hi
            return carry

        lax.fori_loop(0, C_EDGE // 4, mrow, 0)
        pltpu.sync_copy(mb, shared.at[ib.at[q, 1]], add=True)

    for q in range(4):
        start_idx(q, q)
    start_main(0, 0, 0)
    start_main(1, 1, 1)

    def quad(t, carry):
        j0 = 4 * t
        for k in range(4):
            j = j0 + k
            q, b = k, k % 2

            @pl.when(j < my_nch)
            def _():
                compute(j, q, b)

            @pl.when(j + 4 < my_nch)
            def _():
                start_idx(j + 4, q)

            @pl.when(j + 2 < my_nch)
            def _():
                start_main(j + 2, (k + 2) % 4, b)

        return carry

    lax.fori_loop(0, NCH0 // 4 - 1, quad, 0)
    for k in range(4):
        j = NCH0 - 4 + k

        @pl.when(j < my_nch)
        def _():
            compute(j, k, k % 2)

        if k < 2:
            @pl.when(j + 2 < my_nch)
            def _():
                start_main(j + 2, (k + 2) % 4, k % 2)

    plsc.subcore_barrier()

    for r5 in range(RPT // C_EDGE):
        rows = pl.ds(s * RPT + r5 * C_EDGE, C_EDGE)
        pltpu.sync_copy(shared.at[rows], r0)
        pltpu.sync_copy(r0, out_h.at[c, rows])


def _sc_scatter(y, attr, srcm, dstm):
    mesh = plsc.VectorSubcoreMesh(core_axis_name="c", subcore_axis_name="s")
    f = functools.partial(
        pl.kernel,
        out_type=jax.ShapeDtypeStruct((NC, NPAD, DIM), jnp.float32),
        mesh=mesh,
        compiler_params=pltpu.CompilerParams(needs_layout_passes=False),
        scratch_types=[
            pltpu.VMEM((4, 2, C_EDGE), jnp.int32),
            pltpu.VMEM((C_EDGE, DIM), jnp.float32),
            pltpu.VMEM((C_EDGE, DIM), jnp.float32),
            pltpu.VMEM((C_EDGE, DIM // 2), jnp.int32),
            pltpu.VMEM((C_EDGE, DIM // 2), jnp.int32),
            pltpu.VMEM((C_EDGE, DIM), jnp.float32),
            pltpu.VMEM_SHARED((NPAD, DIM), jnp.float32),
            pltpu.SemaphoreType.DMA,
            pltpu.SemaphoreType.DMA,
            pltpu.SemaphoreType.DMA,
            pltpu.SemaphoreType.DMA,
            pltpu.SemaphoreType.DMA,
            pltpu.SemaphoreType.DMA,
            pltpu.SemaphoreType.DMA,
            pltpu.SemaphoreType.DMA,
        ],
    )(_scatter_body)
    return f(y, attr, srcm, dstm)


def _layer_body(x_ref, a_ref, ws_ref, wa_ref, wm_ref, xo_ref, yo_ref):
    agg = a_ref[0] + a_ref[1]
    h = jnp.dot(x_ref[...], ws_ref[...], **_DOT32) + jnp.dot(
        agg, wa_ref[...], **_DOT32)
    x_new = h * jax.nn.sigmoid(h)
    xo_ref[...] = x_new
    if yo_ref is not None:
        yo_ref[...] = jnp.dot(x_new, wm_ref[...], **_DOT32)


def _layer_update(x, agg2, w_self, w_agg, w_msg_next):
    blk = 1000
    grid = N_NODES // blk
    with_y = w_msg_next is not None
    wm = w_msg_next if with_y else jnp.zeros((DIM, DIM), jnp.float32)
    out_shape = [jax.ShapeDtypeStruct((N_NODES, DIM), jnp.float32)]
    out_specs = [pl.BlockSpec((blk, DIM), lambda i: (i, 0))]
    if with_y:
        out_shape.append(jax.ShapeDtypeStruct((N_NODES, DIM), jnp.float32))
        out_specs.append(pl.BlockSpec((blk, DIM), lambda i: (i, 0)))
        body = _layer_body
    else:
        body = lambda x_ref, a_ref, ws, wa, wm_, xo: _layer_body(
            x_ref, a_ref, ws, wa, wm_, xo, None)
    res = pl.pallas_call(
        body,
        grid=(grid,),
        in_specs=[
            pl.BlockSpec((blk, DIM), lambda i: (i, 0)),
            pl.BlockSpec((NC, blk, DIM), lambda i: (0, i, 0)),
            pl.BlockSpec((DIM, DIM), lambda i: (0, 0)),
            pl.BlockSpec((DIM, DIM), lambda i: (0, 0)),
            pl.BlockSpec((DIM, DIM), lambda i: (0, 0)),
        ],
        out_specs=out_specs,
        out_shape=out_shape,
    )(x, agg2, w_self, w_agg, wm)
    return res if with_y else (res[0], None)


def _ln(x, g, b):
    m = jnp.mean(x, axis=-1, keepdims=True)
    v = jnp.mean((x - m) ** 2, axis=-1, keepdims=True)
    return (x - m) / jnp.sqrt(v + 1e-5) * g + b


def _head_body(x_ref, batch_ref, uni_ref,
               pw_ref, pb_ref, uw_ref, ub_ref,
               p2u_wv_ref, p2u_bv_ref, p2u_wo_ref, p2u_bo_ref,
               u2p_wv_ref, u2p_bv_ref, u2p_wo_ref, u2p_bo_ref,
               ln1g_ref, ln1b_ref, ln2g_ref, ln2b_ref,
               ln3g_ref, ln3b_ref,
               f1w_ref, f1b_ref, f2w_ref, f2b_ref,
               p1w_ref, p1b_ref, p2w_ref, p2b_ref,
               out_ref):
    blk = 500
    pooled = jnp.zeros((N_GRAPHS, DIM), jnp.float32)
    gids = lax.broadcasted_iota(jnp.int32, (blk, N_GRAPHS), 1)
    for i in range(N_NODES // blk):
        b = batch_ref[pl.ds(i * blk, blk), :]
        mask = (b == gids).astype(jnp.float32)
        xc = x_ref[pl.ds(i * blk, blk), :]
        pooled = pooled + lax.dot_general(
            mask, xc, (((0,), (0,)), ((), ())), **_DOT32)

    p = _dot_nt(pooled, pw_ref[...]) + pb_ref[...]
    u = _dot_nt(uni_ref[...], uw_ref[...]) + ub_ref[...]

    att_p = _dot_nt(_dot_nt(u, p2u_wv_ref[...]) + p2u_bv_ref[...],
                    p2u_wo_ref[...]) + p2u_bo_ref[...]
    att_u = _dot_nt(_dot_nt(p, u2p_wv_ref[...]) + u2p_bv_ref[...],
                    u2p_wo_ref[...]) + u2p_bo_ref[...]
    pa = _ln(p + att_p, ln1g_ref[...], ln1b_ref[...])
    ua = _ln(u + att_u, ln2g_ref[...], ln2b_ref[...])
    fused = jnp.concatenate([pa, ua], axis=-1)
    h = _dot_nt(jax.nn.relu(_dot_nt(fused, f1w_ref[...]) + f1b_ref[...]),
                f2w_ref[...]) + f2b_ref[...]
    fused = _ln(fused + h, ln3g_ref[...], ln3b_ref[...])
    h2 = jax.nn.relu(_dot_nt(fused, p1w_ref[...]) + p1b_ref[...])
    out_ref[...] = (jnp.sum(h2 * p2w_ref[...], axis=-1, keepdims=True)
                    + p2b_ref[0, 0])


def _pool_head(x, batch, unimol, params):
    p = params
    row = lambda a: a.reshape(1, -1)
    args = (
        x, batch.reshape(N_NODES, 1).astype(jnp.int32), unimol,
        p["pamnet_proj_W"], row(p["pamnet_proj_b"]),
        p["unimol_proj_W"], row(p["unimol_proj_b"]),
        p["p2u"]["Wv"], row(p["p2u"]["bv"]), p["p2u"]["Wo"], row(p["p2u"]["bo"]),
        p["u2p"]["Wv"], row(p["u2p"]["bv"]), p["u2p"]["Wo"], row(p["u2p"]["bo"]),
        row(p["ln1_g"]), row(p["ln1_b"]), row(p["ln2_g"]), row(p["ln2_b"]),
        row(p["ln3_g"]), row(p["ln3_b"]),
        p["ffn_W1"], row(p["ffn_b1"]), p["ffn_W2"], row(p["ffn_b2"]),
        p["pred_W1"], row(p["pred_b1"]), p["pred_W2"], row(p["pred_b2"]),
    )
    out = pl.pallas_call(
        _head_body,
        out_shape=jax.ShapeDtypeStruct((N_GRAPHS, 1), jnp.float32),
    )(*args)
    return out[:, 0]


def kernel(params, x_raw, pos, edge_index, batch, unimol_embeddings):
    src = edge_index[0].astype(jnp.int32)
    dst = edge_index[1].astype(jnp.int32)
    pad = E_PAD - src.shape[0]
    src_flat = jnp.concatenate([src, jnp.zeros((pad,), jnp.int32)])
    dst_flat = jnp.concatenate(
        [dst, jnp.full((pad,), N_NODES, jnp.int32)])
    pos_pad = jnp.concatenate(
        [pos.astype(jnp.float32), jnp.zeros((NPAD - N_NODES, 3), jnp.float32)])
    centers = jnp.linspace(0.0, 5.0, NRBF).astype(jnp.float32)

    src_mat = src_flat.reshape(TOTCH, C_EDGE)
    dst_mat = dst_flat.reshape(TOTCH, C_EDGE)

    dist = _edge_dist2(pos_pad, src_flat, dst_flat)
    attr = _edge_attr(dist, centers, params["W_rbf"], params["b_rbf"])

    x, y = _embed(x_raw, params["embeddings"], params["W_msg"][0])
    for l in range(N_LAYER):
        agg2 = _sc_scatter(y, attr, src_mat, dst_mat)
        wm_next = params["W_msg"][l + 1] if l + 1 < N_LAYER else None
        x, y = _layer_update(x, agg2[:, :N_NODES, :], params["W_self"][l],
                             params["W_agg"][l], wm_next)

    return _pool_head(x, batch, unimol_embeddings, params)


# speedup vs baseline: 1.0039x; 1.0039x over previous
import functools

import jax
import jax.numpy as jnp
from jax import lax
from jax.experimental import pallas as pl
from jax.experimental.pallas import tpu as pltpu
from jax.experimental.pallas import tpu_sc as plsc

N_NODES = 10000
DIM = 128
N_LAYER = 3
N_GRAPHS = 64
NRBF = 16

NC = 2
NS = 16
NW = NC * NS
C_EDGE = 64
NCH0 = 188
NCH1 = 132
TOTCH = NS * (NCH0 + NCH1)
E_PAD = TOTCH * C_EDGE
EPT = E_PAD // NW
NPAD = 10240
RPT = NPAD // NS

_DOT32 = dict(preferred_element_type=jnp.float32)


_IDX_LO = jnp.asarray([w + 16 * (w // 16) for w in range(DIM // 2)])
_IDX_HI = _IDX_LO + 16


def _bf16_bits(x):
    u = lax.bitcast_convert_type(x, jnp.int32)
    return lax.shift_right_logical(u + 0x8000, 16)


def _dot_nt(a, b):
    return lax.dot_general(a, b, (((1,), (1,)), ((), ())), **_DOT32)


def _embed_body(xr_ref, emb_ref, w0_ref, x_ref, y_ref):
    ids = xr_ref[...]
    cls = lax.broadcasted_iota(jnp.int32, (ids.shape[0], emb_ref.shape[0]), 1)
    onehot = (ids == cls).astype(jnp.float32)
    x = jnp.dot(onehot, emb_ref[...], **_DOT32)
    x_ref[...] = x
    y_ref[...] = jnp.dot(x, w0_ref[...], **_DOT32)


def _embed(x_raw, emb, w_msg0):
    blk = 1000
    grid = N_NODES // blk
    return pl.pallas_call(
        _embed_body,
        grid=(grid,),
        in_specs=[
            pl.BlockSpec((blk, 1), lambda i: (i, 0)),
            pl.BlockSpec(emb.shape, lambda i: (0, 0)),
            pl.BlockSpec((DIM, DIM), lambda i: (0, 0)),
        ],
        out_specs=[
            pl.BlockSpec((blk, DIM), lambda i: (i, 0)),
            pl.BlockSpec((blk, DIM), lambda i: (i, 0)),
        ],
        out_shape=[
            jax.ShapeDtypeStruct((N_NODES, DIM), jnp.float32),
            jax.ShapeDtypeStruct((N_NODES, DIM), jnp.float32),
        ],
    )(x_raw.reshape(N_NODES, 1).astype(jnp.int32), emb, w_msg0)


def _dist2_body(px_h, py_h, pz_h, src_h, dst_h, out_h,
                px, py, pz, sv, dv, ov):
    wid = lax.axis_index("c") * NS + lax.axis_index("s")
    base = wid * EPT
    pltpu.sync_copy(px_h, px)
    pltpu.sync_copy(py_h, py)
    pltpu.sync_copy(pz_h, pz)
    pltpu.sync_copy(src_h.at[pl.ds(base, EPT)], sv)
    pltpu.sync_copy(dst_h.at[pl.ds(base, EPT)], dv)

    def body(k, carry):
        off = k * 16
        si = sv[pl.ds(off, 16)]
        di = dv[pl.ds(off, 16)]
        dx = plsc.load_gather(px, [di]) - plsc.load_gather(px, [si])
        dy = plsc.load_gather(py, [di]) - plsc.load_gather(py, [si])
        dz = plsc.load_gather(pz, [di]) - plsc.load_gather(pz, [si])
        ov[pl.ds(off, 16)] = dx * dx + dy * dy + dz * dz
        return carry

    lax.fori_loop(0, EPT // 16, body, 0)
    pltpu.sync_copy(ov, out_h.at[pl.ds(base, EPT)])


def _edge_dist2(pos_pad, src_flat, dst_flat):
    mesh = plsc.VectorSubcoreMesh(core_axis_name="c", subcore_axis_name="s")
    f = functools.partial(
        pl.kernel,
        out_type=jax.ShapeDtypeStruct((E_PAD,), jnp.float32),
        mesh=mesh,
        compiler_params=pltpu.CompilerParams(needs_layout_passes=False),
        scratch_types=[
            pltpu.VMEM((NPAD,), jnp.float32),
            pltpu.VMEM((NPAD,), jnp.float32),
            pltpu.VMEM((NPAD,), jnp.float32),
            pltpu.VMEM((EPT,), jnp.int32),
            pltpu.VMEM((EPT,), jnp.int32),
            pltpu.VMEM((EPT,), jnp.float32),
        ],
    )(_dist2_body)
    return f(pos_pad[:, 0], pos_pad[:, 1], pos_pad[:, 2], src_flat, dst_flat)


def _attr_body(d2_ref, cen_ref, wlo_ref, blo_ref, whi_ref, bhi_ref, out_ref):
    dist = jnp.sqrt(d2_ref[...] + 1e-12)
    z = dist - cen_ref[...]
    rbf = jnp.exp(-(z * z))
    ulo = lax.bitcast_convert_type(
        jnp.dot(rbf, wlo_ref[...], **_DOT32) + blo_ref[...], jnp.int32)
    uhi = lax.bitcast_convert_type(
        jnp.dot(rbf, whi_ref[...], **_DOT32) + bhi_ref[...], jnp.int32)
    out_ref[...] = lax.bitwise_or(
        lax.bitwise_and(uhi + 0x8000, -65536),
        lax.shift_right_logical(ulo + 0x8000, 16))


def _edge_attr(dist2, centers, w_rbf, b_rbf):
    blk = 4096
    grid = E_PAD // blk
    return pl.pallas_call(
        _attr_body,
        grid=(grid,),
        in_specs=[
            pl.BlockSpec((blk, 1), lambda i: (i, 0)),
            pl.BlockSpec((1, NRBF), lambda i: (0, 0)),
            pl.BlockSpec((NRBF, DIM // 2), lambda i: (0, 0)),
            pl.BlockSpec((1, DIM // 2), lambda i: (0, 0)),
            pl.BlockSpec((NRBF, DIM // 2), lambda i: (0, 0)),
            pl.BlockSpec((1, DIM // 2), lambda i: (0, 0)),
        ],
        out_specs=pl.BlockSpec((blk, DIM // 2), lambda i: (i, 0)),
        out_shape=jax.ShapeDtypeStruct((E_PAD, DIM // 2), jnp.int32),
    )(dist2.reshape(E_PAD, 1), centers.reshape(1, NRBF),
      w_rbf[:, _IDX_LO], b_rbf[_IDX_LO].reshape(1, DIM // 2),
      w_rbf[:, _IDX_HI], b_rbf[_IDX_HI].reshape(1, DIM // 2))


def _scatter_body(y_h, attr_h, srcm_h, dstm_h, out_h,
                  ib, r0, r1, a0, a1, m0, shared,
                  si0, si1, si2, si3, sg0, sg1, sa0, sa1):
    c = lax.axis_index("c")
    s = lax.axis_index("s")
    base_ch = jnp.where(c == 0, s * NCH0, NS * NCH0 + s * NCH1)
    my_nch = jnp.where(c == 0, NCH0, NCH1)
    rbufs = (r0, r1)
    abufs = (a0, a1)
    mbufs = (m0, m0)
    sgs = (sg0, sg1)
    sas = (sa0, sa1)
    sis = (si0, si1, si2, si3)

    zv = jnp.zeros((16,), jnp.float32)

    def zrow(r, carry):
        for cc in range(DIM // 16):
            r0[r, pl.ds(cc * 16, 16)] = zv
        return carry

    lax.fori_loop(0, C_EDGE, zrow, 0)
    for r5 in range(RPT // C_EDGE):
        pltpu.sync_copy(r0, shared.at[pl.ds(s * RPT + r5 * C_EDGE, C_EDGE)])
    plsc.subcore_barrier()

    def start_idx(j, q):
        g = base_ch + j
        pltpu.make_async_copy(srcm_h.at[g], ib.at[q, 0], sis[q]).start()
        pltpu.make_async_copy(dstm_h.at[g], ib.at[q, 1], sis[q]).start()

    def start_main(j, q, b):
        g = base_ch + j
        pltpu.make_async_copy(srcm_h.at[g], ib.at[q, 0], sis[q]).wait()
        pltpu.make_async_copy(dstm_h.at[g], ib.at[q, 1], sis[q]).wait()
        pltpu.make_async_copy(y_h.at[ib.at[q, 0]], rbufs[b], sgs[b]).start()
        pltpu.make_async_copy(
            attr_h.at[pl.ds((base_ch + j) * C_EDGE, C_EDGE)],
            abufs[b], sas[b]).start()

    def compute(j, q, b):
        rb, ab, mb = rbufs[b], abufs[b], mbufs[b]
        pltpu.make_async_copy(y_h.at[ib.at[q, 0]], rb, sgs[b]).wait()
        pltpu.make_async_copy(
            attr_h.at[pl.ds((base_ch + j) * C_EDGE, C_EDGE)], ab,
            sas[b]).wait()

        hmask = jnp.full((16,), -65536, jnp.int32)

        def mrow(r4, carry):
            for rr in range(4):
                r = 4 * r4 + rr
                for cc in range(DIM // 32):
                    a32 = ab[r, pl.ds(cc * 16, 16)]
                    lo = plsc.bitcast(lax.shift_left(a32, 16), jnp.float32)
                    hi = plsc.bitcast(lax.bitwise_and(a32, hmask), jnp.float32)
                    sl0 = pl.ds(cc * 32, 16)
                    sl1 = pl.ds(cc * 32 + 16, 16)
                    mb[r, sl0] = rb[r, sl0] * lo
                    mb[r, sl1] = rb[r, sl1] * hi
            return carry

        lax.fori_loop(0, C_EDGE // 4, mrow, 0)
        pltpu.sync_copy(mb, shared.at[ib.at[q, 1]], add=True)

    for q in range(4):
        start_idx(q, q)
    start_main(0, 0, 0)
    start_main(1, 1, 1)

    def quad(t, carry):
        j0 = 4 * t
        for k in range(4):
            j = j0 + k
            q, b = k, k % 2

            @pl.when(j < my_nch)
            def _():
                compute(j, q, b)

            @pl.when(j + 4 < my_nch)
            def _():
                start_idx(j + 4, q)

            @pl.when(j + 2 < my_nch)
            def _():
                start_main(j + 2, (k + 2) % 4, b)

        return carry

    lax.fori_loop(0, NCH0 // 4 - 1, quad, 0)
    for k in range(4):
        j = NCH0 - 4 + k

        @pl.when(j < my_nch)
        def _():
            compute(j, k, k % 2)

        if k < 2:
            @pl.when(j + 2 < my_nch)
            def _():
                start_main(j + 2, (k + 2) % 4, k % 2)

    plsc.subcore_barrier()

    for r5 in range(RPT // C_EDGE):
        rows = pl.ds(s * RPT + r5 * C_EDGE, C_EDGE)
        pltpu.sync_copy(shared.at[rows], r0)
        pltpu.sync_copy(r0, out_h.at[c, rows])


def _sc_scatter(y, attr, srcm, dstm):
    mesh = plsc.VectorSubcoreMesh(core_axis_name="c", subcore_axis_name="s")
    f = functools.partial(
        pl.kernel,
        out_type=jax.ShapeDtypeStruct((NC, NPAD, DIM), jnp.float32),
        mesh=mesh,
        compiler_params=pltpu.CompilerParams(needs_layout_passes=False),
        scratch_types=[
            pltpu.VMEM((4, 2, C_EDGE), jnp.int32),
            pltpu.VMEM((C_EDGE, DIM), jnp.float32),
            pltpu.VMEM((C_EDGE, DIM), jnp.float32),
            pltpu.VMEM((C_EDGE, DIM // 2), jnp.int32),
            pltpu.VMEM((C_EDGE, DIM // 2), jnp.int32),
            pltpu.VMEM((C_EDGE, DIM), jnp.float32),
            pltpu.VMEM_SHARED((NPAD, DIM), jnp.float32),
            pltpu.SemaphoreType.DMA,
            pltpu.SemaphoreType.DMA,
            pltpu.SemaphoreType.DMA,
            pltpu.SemaphoreType.DMA,
            pltpu.SemaphoreType.DMA,
            pltpu.SemaphoreType.DMA,
            pltpu.SemaphoreType.DMA,
            pltpu.SemaphoreType.DMA,
        ],
    )(_scatter_body)
    return f(y, attr, srcm, dstm)


def _layer_body(x_ref, a_ref, ws_ref, wa_ref, wm_ref, xo_ref, yo_ref):
    agg = a_ref[0] + a_ref[1]
    h = jnp.dot(x_ref[...], ws_ref[...], **_DOT32) + jnp.dot(
        agg, wa_ref[...], **_DOT32)
    x_new = h * jax.nn.sigmoid(h)
    xo_ref[...] = x_new
    if yo_ref is not None:
        yo_ref[...] = jnp.dot(x_new, wm_ref[...], **_DOT32)


def _layer_update(x, agg2, w_self, w_agg, w_msg_next):
    blk = 1000
    grid = N_NODES // blk
    with_y = w_msg_next is not None
    wm = w_msg_next if with_y else jnp.zeros((DIM, DIM), jnp.float32)
    out_shape = [jax.ShapeDtypeStruct((N_NODES, DIM), jnp.float32)]
    out_specs = [pl.BlockSpec((blk, DIM), lambda i: (i, 0))]
    if with_y:
        out_shape.append(jax.ShapeDtypeStruct((N_NODES, DIM), jnp.float32))
        out_specs.append(pl.BlockSpec((blk, DIM), lambda i: (i, 0)))
        body = _layer_body
    else:
        body = lambda x_ref, a_ref, ws, wa, wm_, xo: _layer_body(
            x_ref, a_ref, ws, wa, wm_, xo, None)
    res = pl.pallas_call(
        body,
        grid=(grid,),
        in_specs=[
            pl.BlockSpec((blk, DIM), lambda i: (i, 0)),
            pl.BlockSpec((NC, blk, DIM), lambda i: (0, i, 0)),
            pl.BlockSpec((DIM, DIM), lambda i: (0, 0)),
            pl.BlockSpec((DIM, DIM), lambda i: (0, 0)),
            pl.BlockSpec((DIM, DIM), lambda i: (0, 0)),
        ],
        out_specs=out_specs,
        out_shape=out_shape,
    )(x, agg2, w_self, w_agg, wm)
    return res if with_y else (res[0], None)


def _ln(x, g, b):
    m = jnp.mean(x, axis=-1, keepdims=True)
    v = jnp.mean((x - m) ** 2, axis=-1, keepdims=True)
    return (x - m) / jnp.sqrt(v + 1e-5) * g + b


def _head_body(x_ref, batch_ref, uni_ref,
               pw_ref, pb_ref, uw_ref, ub_ref,
               p2u_wv_ref, p2u_bv_ref, p2u_wo_ref, p2u_bo_ref,
               u2p_wv_ref, u2p_bv_ref, u2p_wo_ref, u2p_bo_ref,
               ln1g_ref, ln1b_ref, ln2g_ref, ln2b_ref,
               ln3g_ref, ln3b_ref,
               f1w_ref, f1b_ref, f2w_ref, f2b_ref,
               p1w_ref, p1b_ref, p2w_ref, p2b_ref,
               out_ref):
    blk = 500
    pooled = jnp.zeros((N_GRAPHS, DIM), jnp.float32)
    gids = lax.broadcasted_iota(jnp.int32, (blk, N_GRAPHS), 1)
    for i in range(N_NODES // blk):
        b = batch_ref[pl.ds(i * blk, blk), :]
        mask = (b == gids).astype(jnp.float32)
        xc = x_ref[pl.ds(i * blk, blk), :]
        pooled = pooled + lax.dot_general(
            mask, xc, (((0,), (0,)), ((), ())), **_DOT32)

    p = _dot_nt(pooled, pw_ref[...]) + pb_ref[...]
    u = _dot_nt(uni_ref[...], uw_ref[...]) + ub_ref[...]

    att_p = _dot_nt(_dot_nt(u, p2u_wv_ref[...]) + p2u_bv_ref[...],
                    p2u_wo_ref[...]) + p2u_bo_ref[...]
    att_u = _dot_nt(_dot_nt(p, u2p_wv_ref[...]) + u2p_bv_ref[...],
                    u2p_wo_ref[...]) + u2p_bo_ref[...]
    pa = _ln(p + att_p, ln1g_ref[...], ln1b_ref[...])
    ua = _ln(u + att_u, ln2g_ref[...], ln2b_ref[...])
    fused = jnp.concatenate([pa, ua], axis=-1)
    h = _dot_nt(jax.nn.relu(_dot_nt(fused, f1w_ref[...]) + f1b_ref[...]),
                f2w_ref[...]) + f2b_ref[...]
    fused = _ln(fused + h, ln3g_ref[...], ln3b_ref[...])
    h2 = jax.nn.relu(_dot_nt(fused, p1w_ref[...]) + p1b_ref[...])
    out_ref[...] = (jnp.sum(h2 * p2w_ref[...], axis=-1, keepdims=True)
                    + p2b_ref[0, 0])


def _pool_head(x, batch, unimol, params):
    p = params
    row = lambda a: a.reshape(1, -1)
    args = (
        x, batch.reshape(N_NODES, 1).astype(jnp.int32), unimol,
        p["pamnet_proj_W"], row(p["pamnet_proj_b"]),
        p["unimol_proj_W"], row(p["unimol_proj_b"]),
        p["p2u"]["Wv"], row(p["p2u"]["bv"]), p["p2u"]["Wo"], row(p["p2u"]["bo"]),
        p["u2p"]["Wv"], row(p["u2p"]["bv"]), p["u2p"]["Wo"], row(p["u2p"]["bo"]),
        row(p["ln1_g"]), row(p["ln1_b"]), row(p["ln2_g"]), row(p["ln2_b"]),
        row(p["ln3_g"]), row(p["ln3_b"]),
        p["ffn_W1"], row(p["ffn_b1"]), p["ffn_W2"], row(p["ffn_b2"]),
        p["pred_W1"], row(p["pred_b1"]), p["pred_W2"], row(p["pred_b2"]),
    )
    out = pl.pallas_call(
        _head_body,
        out_shape=jax.ShapeDtypeStruct((N_GRAPHS, 1), jnp.float32),
    )(*args)
    return out[:, 0]


def kernel(params, x_raw, pos, edge_index, batch, unimol_embeddings):
    src = edge_index[0].astype(jnp.int32)
    dst = edge_index[1].astype(jnp.int32)
    pad = E_PAD - src.shape[0]
    src_flat = jnp.concatenate([src, jnp.zeros((pad,), jnp.int32)])
    dst_flat = jnp.concatenate(
        [dst, jnp.full((pad,), N_NODES, jnp.int32)])
    pos_pad = jnp.concatenate(
        [pos.astype(jnp.float32), jnp.zeros((NPAD - N_NODES, 3), jnp.float32)])
    centers = jnp.linspace(0.0, 5.0, NRBF).astype(jnp.float32)

    src_mat = src_flat.reshape(TOTCH, C_EDGE)
    dst_mat = dst_flat.reshape(TOTCH, C_EDGE)

    dist2 = _edge_dist2(pos_pad, src_flat, dst_flat)
    attr = _edge_attr(dist2, centers, params["W_rbf"], params["b_rbf"])

    x, y = _embed(x_raw, params["embeddings"], params["W_msg"][0])
    for l in range(N_LAYER):
        agg2 = _sc_scatter(y, attr, src_mat, dst_mat)
        wm_next = params["W_msg"][l + 1] if l + 1 < N_LAYER else None
        x, y = _layer_update(x, agg2[:, :N_NODES, :], params["W_self"][l],
                             params["W_agg"][l], wm_next)

    return _pool_head(x, batch, unimol_embeddings, params)


# speedup vs baseline: 1.7512x; 1.7445x over previous
import functools

import jax
import jax.numpy as jnp
from jax import lax
from jax.experimental import pallas as pl
from jax.experimental.pallas import tpu as pltpu
from jax.experimental.pallas import tpu_sc as plsc

N_NODES = 10000
DIM = 128
N_LAYER = 3
N_GRAPHS = 64
NRBF = 16

NC = 2
NS = 16
NW = NC * NS
C_EDGE = 64
NCH0 = 192
NCH1 = 122
TOTCH = NS * (NCH0 + NCH1)
E_PAD = TOTCH * C_EDGE
EPT = E_PAD // NW
NPAD = 10240
RPT = NPAD // NS

_DOT32 = dict(preferred_element_type=jnp.float32)


_IDX_LO = jnp.asarray([w + 16 * (w // 16) for w in range(DIM // 2)])
_IDX_HI = _IDX_LO + 16


def _bf16_bits(x):
    u = lax.bitcast_convert_type(x, jnp.int32)
    return lax.shift_right_logical(u + 0x8000, 16)


def _dot_nt(a, b):
    return lax.dot_general(a, b, (((1,), (1,)), ((), ())), **_DOT32)


def _embed_body(xr_ref, emb_ref, w0_ref, x_ref, y_ref):
    ids = xr_ref[...]
    cls = lax.broadcasted_iota(jnp.int32, (ids.shape[0], emb_ref.shape[0]), 1)
    onehot = (ids == cls).astype(jnp.float32)
    x = jnp.dot(onehot, emb_ref[...], **_DOT32)
    x_ref[...] = x
    y_ref[...] = jnp.dot(x, w0_ref[...], **_DOT32)


def _embed(x_raw, emb, w_msg0):
    blk = 1000
    grid = N_NODES // blk
    return pl.pallas_call(
        _embed_body,
        grid=(grid,),
        in_specs=[
            pl.BlockSpec((blk, 1), lambda i: (i, 0)),
            pl.BlockSpec(emb.shape, lambda i: (0, 0)),
            pl.BlockSpec((DIM, DIM), lambda i: (0, 0)),
        ],
        out_specs=[
            pl.BlockSpec((blk, DIM), lambda i: (i, 0)),
            pl.BlockSpec((blk, DIM), lambda i: (i, 0)),
        ],
        out_shape=[
            jax.ShapeDtypeStruct((N_NODES, DIM), jnp.float32),
            jax.ShapeDtypeStruct((N_NODES, DIM), jnp.float32),
        ],
    )(x_raw.reshape(N_NODES, 1).astype(jnp.int32), emb, w_msg0)


def _dist2_body(px_h, py_h, pz_h, src_h, dst_h, out_h,
                px, py, pz, sv, dv, ov):
    wid = lax.axis_index("c") * NS + lax.axis_index("s")
    base = wid * EPT
    pltpu.sync_copy(px_h, px)
    pltpu.sync_copy(py_h, py)
    pltpu.sync_copy(pz_h, pz)
    pltpu.sync_copy(src_h.at[pl.ds(base, EPT)], sv)
    pltpu.sync_copy(dst_h.at[pl.ds(base, EPT)], dv)

    def body(k, carry):
        off = k * 16
        si = sv[pl.ds(off, 16)]
        di = dv[pl.ds(off, 16)]
        dx = plsc.load_gather(px, [di]) - plsc.load_gather(px, [si])
        dy = plsc.load_gather(py, [di]) - plsc.load_gather(py, [si])
        dz = plsc.load_gather(pz, [di]) - plsc.load_gather(pz, [si])
        ov[pl.ds(off, 16)] = dx * dx + dy * dy + dz * dz
        return carry

    lax.fori_loop(0, EPT // 16, body, 0)
    pltpu.sync_copy(ov, out_h.at[pl.ds(base, EPT)])


def _edge_dist2(pos_pad, src_flat, dst_flat):
    mesh = plsc.VectorSubcoreMesh(core_axis_name="c", subcore_axis_name="s")
    f = functools.partial(
        pl.kernel,
        out_type=jax.ShapeDtypeStruct((E_PAD,), jnp.float32),
        mesh=mesh,
        compiler_params=pltpu.CompilerParams(needs_layout_passes=False),
        scratch_types=[
            pltpu.VMEM((NPAD,), jnp.float32),
            pltpu.VMEM((NPAD,), jnp.float32),
            pltpu.VMEM((NPAD,), jnp.float32),
            pltpu.VMEM((EPT,), jnp.int32),
            pltpu.VMEM((EPT,), jnp.int32),
            pltpu.VMEM((EPT,), jnp.float32),
        ],
    )(_dist2_body)
    return f(pos_pad[:, 0], pos_pad[:, 1], pos_pad[:, 2], src_flat, dst_flat)


def _attr_body(d2_ref, cen_ref, wlo_ref, blo_ref, whi_ref, bhi_ref, out_ref):
    dist = jnp.sqrt(d2_ref[...] + 1e-12)
    z = dist - cen_ref[...]
    rbf = jnp.exp(-(z * z))
    ulo = lax.bitcast_convert_type(
        jnp.dot(rbf, wlo_ref[...], **_DOT32) + blo_ref[...], jnp.int32)
    uhi = lax.bitcast_convert_type(
        jnp.dot(rbf, whi_ref[...], **_DOT32) + bhi_ref[...], jnp.int32)
    out_ref[...] = lax.bitwise_or(
        lax.bitwise_and(uhi + 0x8000, -65536),
        lax.shift_right_logical(ulo + 0x8000, 16))


def _edge_attr(dist2, centers, w_rbf, b_rbf):
    blk = 2048
    grid = E_PAD // blk
    return pl.pallas_call(
        _attr_body,
        grid=(grid,),
        in_specs=[
            pl.BlockSpec((blk, 1), lambda i: (i, 0)),
            pl.BlockSpec((1, NRBF), lambda i: (0, 0)),
            pl.BlockSpec((NRBF, DIM // 2), lambda i: (0, 0)),
            pl.BlockSpec((1, DIM // 2), lambda i: (0, 0)),
            pl.BlockSpec((NRBF, DIM // 2), lambda i: (0, 0)),
            pl.BlockSpec((1, DIM // 2), lambda i: (0, 0)),
        ],
        out_specs=pl.BlockSpec((blk, DIM // 2), lambda i: (i, 0)),
        out_shape=jax.ShapeDtypeStruct((E_PAD, DIM // 2), jnp.int32),
    )(dist2.reshape(E_PAD, 1), centers.reshape(1, NRBF),
      w_rbf[:, _IDX_LO], b_rbf[_IDX_LO].reshape(1, DIM // 2),
      w_rbf[:, _IDX_HI], b_rbf[_IDX_HI].reshape(1, DIM // 2))


def _scatter_body(y_h, attr_h, ep_h, out_h,
                  ib, r0, r1, a0, a1, m0, shared,
                  si0, si1, si2, si3, sg0, sg1, sa0, sa1):
    c = lax.axis_index("c")
    s = lax.axis_index("s")
    base_ch = jnp.where(c == 0, s * NCH0, NS * NCH0 + s * NCH1)
    my_nch = jnp.where(c == 0, NCH0, NCH1)
    rbufs = (r0, r1)
    abufs = (a0, a1)
    mbufs = (m0, m0)
    sgs = (sg0, sg1)
    sas = (sa0, sa1)
    sis = (si0, si1, si2, si3)

    zv = jnp.zeros((16,), jnp.float32)

    def zrow(r, carry):
        for cc in range(DIM // 16):
            r0[r, pl.ds(cc * 16, 16)] = zv
        return carry

    lax.fori_loop(0, C_EDGE, zrow, 0)
    for r5 in range(RPT // C_EDGE):
        pltpu.sync_copy(r0, shared.at[pl.ds(s * RPT + r5 * C_EDGE, C_EDGE)])
    plsc.subcore_barrier()

    def start_idx(j, q):
        pltpu.make_async_copy(ep_h.at[base_ch + j], ib.at[q], sis[q]).start()

    def start_main(j, q, b):
        pltpu.make_async_copy(ep_h.at[base_ch + j], ib.at[q], sis[q]).wait()
        pltpu.make_async_copy(y_h.at[ib.at[q, 0]], rbufs[b], sgs[b]).start()
        pltpu.make_async_copy(
            attr_h.at[pl.ds((base_ch + j) * C_EDGE, C_EDGE)],
            abufs[b], sas[b]).start()

    def compute(j, q, b):
        rb, ab, mb = rbufs[b], abufs[b], mbufs[b]
        pltpu.make_async_copy(y_h.at[ib.at[q, 0]], rb, sgs[b]).wait()
        pltpu.make_async_copy(
            attr_h.at[pl.ds((base_ch + j) * C_EDGE, C_EDGE)], ab,
            sas[b]).wait()

        hmask = jnp.full((16,), -65536, jnp.int32)

        def mrow(r4, carry):
            for rr in range(4):
                r = 4 * r4 + rr
                for cc in range(DIM // 32):
                    a32 = ab[r, pl.ds(cc * 16, 16)]
                    lo = plsc.bitcast(lax.shift_left(a32, 16), jnp.float32)
                    hi = plsc.bitcast(lax.bitwise_and(a32, hmask), jnp.float32)
                    sl0 = pl.ds(cc * 32, 16)
                    sl1 = pl.ds(cc * 32 + 16, 16)
                    mb[r, sl0] = rb[r, sl0] * lo
                    mb[r, sl1] = rb[r, sl1] * hi
            return carry

        lax.fori_loop(0, C_EDGE // 4, mrow, 0)
        pltpu.sync_copy(mb, shared.at[ib.at[q, 1]], add=True)

    for q in range(4):
        start_idx(q, q)
    start_main(0, 0, 0)
    start_main(1, 1, 1)

    def quad(t, carry):
        j0 = 4 * t
        for k in range(4):
            j = j0 + k
            q, b = k, k % 2

            @pl.when(j < my_nch)
            def _():
                compute(j, q, b)

            @pl.when(j + 4 < my_nch)
            def _():
                start_idx(j + 4, q)

            @pl.when(j + 2 < my_nch)
            def _():
                start_main(j + 2, (k + 2) % 4, b)

        return carry

    lax.fori_loop(0, NCH0 // 4 - 1, quad, 0)
    for k in range(4):
        j = NCH0 - 4 + k

        @pl.when(j < my_nch)
        def _():
            compute(j, k, k % 2)

        if k < 2:
            @pl.when(j + 2 < my_nch)
            def _():
                start_main(j + 2, (k + 2) % 4, k % 2)

    plsc.subcore_barrier()

    for r5 in range(RPT // C_EDGE):
        rows = pl.ds(s * RPT + r5 * C_EDGE, C_EDGE)
        pltpu.sync_copy(shared.at[rows], r0)
        pltpu.sync_copy(r0, out_h.at[c, rows])


def _sc_scatter(y, attr, ep):
    mesh = plsc.VectorSubcoreMesh(core_axis_name="c", subcore_axis_name="s")
    f = functools.partial(
        pl.kernel,
        out_type=jax.ShapeDtypeStruct((NC, NPAD, DIM), jnp.float32),
        mesh=mesh,
        compiler_params=pltpu.CompilerParams(needs_layout_passes=False),
        scratch_types=[
            pltpu.VMEM((4, 2, C_EDGE), jnp.int32),
            pltpu.VMEM((C_EDGE, DIM), jnp.float32),
            pltpu.VMEM((C_EDGE, DIM), jnp.float32),
            pltpu.VMEM((C_EDGE, DIM // 2), jnp.int32),
            pltpu.VMEM((C_EDGE, DIM // 2), jnp.int32),
            pltpu.VMEM((C_EDGE, DIM), jnp.float32),
            pltpu.VMEM_SHARED((NPAD, DIM), jnp.float32),
            pltpu.SemaphoreType.DMA,
            pltpu.SemaphoreType.DMA,
            pltpu.SemaphoreType.DMA,
            pltpu.SemaphoreType.DMA,
            pltpu.SemaphoreType.DMA,
            pltpu.SemaphoreType.DMA,
            pltpu.SemaphoreType.DMA,
            pltpu.SemaphoreType.DMA,
        ],
    )(_scatter_body)
    return f(y, attr, ep)


def _layer_body(x_ref, a_ref, ws_ref, wa_ref, wm_ref, xo_ref, yo_ref):
    agg = a_ref[0] + a_ref[1]
    h = jnp.dot(x_ref[...], ws_ref[...], **_DOT32) + jnp.dot(
        agg, wa_ref[...], **_DOT32)
    x_new = h * jax.nn.sigmoid(h)
    xo_ref[...] = x_new
    if yo_ref is not None:
        yo_ref[...] = jnp.dot(x_new, wm_ref[...], **_DOT32)


def _layer_update(x, agg2, w_self, w_agg, w_msg_next):
    blk = 1000
    grid = N_NODES // blk
    with_y = w_msg_next is not None
    wm = w_msg_next if with_y else jnp.zeros((DIM, DIM), jnp.float32)
    out_shape = [jax.ShapeDtypeStruct((N_NODES, DIM), jnp.float32)]
    out_specs = [pl.BlockSpec((blk, DIM), lambda i: (i, 0))]
    if with_y:
        out_shape.append(jax.ShapeDtypeStruct((N_NODES, DIM), jnp.float32))
        out_specs.append(pl.BlockSpec((blk, DIM), lambda i: (i, 0)))
        body = _layer_body
    else:
        body = lambda x_ref, a_ref, ws, wa, wm_, xo: _layer_body(
            x_ref, a_ref, ws, wa, wm_, xo, None)
    res = pl.pallas_call(
        body,
        grid=(grid,),
        in_specs=[
            pl.BlockSpec((blk, DIM), lambda i: (i, 0)),
            pl.BlockSpec((NC, blk, DIM), lambda i: (0, i, 0)),
            pl.BlockSpec((DIM, DIM), lambda i: (0, 0)),
            pl.BlockSpec((DIM, DIM), lambda i: (0, 0)),
            pl.BlockSpec((DIM, DIM), lambda i: (0, 0)),
        ],
        out_specs=out_specs,
        out_shape=out_shape,
    )(x, agg2, w_self, w_agg, wm)
    return res if with_y else (res[0], None)


def _ln(x, g, b):
    m = jnp.mean(x, axis=-1, keepdims=True)
    v = jnp.mean((x - m) ** 2, axis=-1, keepdims=True)
    return (x - m) / jnp.sqrt(v + 1e-5) * g + b


def _head_body(x_ref, batch_ref, uni_ref,
               pw_ref, pb_ref, uw_ref, ub_ref,
               p2u_wv_ref, p2u_bv_ref, p2u_wo_ref, p2u_bo_ref,
               u2p_wv_ref, u2p_bv_ref, u2p_wo_ref, u2p_bo_ref,
               ln1g_ref, ln1b_ref, ln2g_ref, ln2b_ref,
               ln3g_ref, ln3b_ref,
               f1w_ref, f1b_ref, f2w_ref, f2b_ref,
               p1w_ref, p1b_ref, p2w_ref, p2b_ref,
               out_ref):
    blk = 500
    pooled = jnp.zeros((N_GRAPHS, DIM), jnp.float32)
    gids = lax.broadcasted_iota(jnp.int32, (blk, N_GRAPHS), 1)
    for i in range(N_NODES // blk):
        b = batch_ref[pl.ds(i * blk, blk), :]
        mask = (b == gids).astype(jnp.float32)
        xc = x_ref[pl.ds(i * blk, blk), :]
        pooled = pooled + lax.dot_general(
            mask, xc, (((0,), (0,)), ((), ())), **_DOT32)

    p = _dot_nt(pooled, pw_ref[...]) + pb_ref[...]
    u = _dot_nt(uni_ref[...], uw_ref[...]) + ub_ref[...]

    att_p = _dot_nt(_dot_nt(u, p2u_wv_ref[...]) + p2u_bv_ref[...],
                    p2u_wo_ref[...]) + p2u_bo_ref[...]
    att_u = _dot_nt(_dot_nt(p, u2p_wv_ref[...]) + u2p_bv_ref[...],
                    u2p_wo_ref[...]) + u2p_bo_ref[...]
    pa = _ln(p + att_p, ln1g_ref[...], ln1b_ref[...])
    ua = _ln(u + att_u, ln2g_ref[...], ln2b_ref[...])
    fused = jnp.concatenate([pa, ua], axis=-1)
    h = _dot_nt(jax.nn.relu(_dot_nt(fused, f1w_ref[...]) + f1b_ref[...]),
                f2w_ref[...]) + f2b_ref[...]
    fused = _ln(fused + h, ln3g_ref[...], ln3b_ref[...])
    h2 = jax.nn.relu(_dot_nt(fused, p1w_ref[...]) + p1b_ref[...])
    out_ref[...] = (jnp.sum(h2 * p2w_ref[...], axis=-1, keepdims=True)
                    + p2b_ref[0, 0])


def _pool_head(x, batch, unimol, params):
    p = params
    row = lambda a: a.reshape(1, -1)
    args = (
        x, batch.reshape(N_NODES, 1).astype(jnp.int32), unimol,
        p["pamnet_proj_W"], row(p["pamnet_proj_b"]),
        p["unimol_proj_W"], row(p["unimol_proj_b"]),
        p["p2u"]["Wv"], row(p["p2u"]["bv"]), p["p2u"]["Wo"], row(p["p2u"]["bo"]),
        p["u2p"]["Wv"], row(p["u2p"]["bv"]), p["u2p"]["Wo"], row(p["u2p"]["bo"]),
        row(p["ln1_g"]), row(p["ln1_b"]), row(p["ln2_g"]), row(p["ln2_b"]),
        row(p["ln3_g"]), row(p["ln3_b"]),
        p["ffn_W1"], row(p["ffn_b1"]), p["ffn_W2"], row(p["ffn_b2"]),
        p["pred_W1"], row(p["pred_b1"]), p["pred_W2"], row(p["pred_b2"]),
    )
    out = pl.pallas_call(
        _head_body,
        out_shape=jax.ShapeDtypeStruct((N_GRAPHS, 1), jnp.float32),
    )(*args)
    return out[:, 0]


def kernel(params, x_raw, pos, edge_index, batch, unimol_embeddings):
    src = edge_index[0].astype(jnp.int32)
    dst = edge_index[1].astype(jnp.int32)
    pad = E_PAD - src.shape[0]
    src_flat = jnp.concatenate([src, jnp.zeros((pad,), jnp.int32)])
    dst_flat = jnp.concatenate(
        [dst, jnp.full((pad,), N_NODES, jnp.int32)])
    pos_pad = jnp.concatenate(
        [pos.astype(jnp.float32), jnp.zeros((NPAD - N_NODES, 3), jnp.float32)])
    centers = jnp.linspace(0.0, 5.0, NRBF).astype(jnp.float32)

    ep = jnp.stack([src_flat.reshape(TOTCH, C_EDGE),
                    dst_flat.reshape(TOTCH, C_EDGE)], axis=1)

    dist2 = _edge_dist2(pos_pad, src_flat, dst_flat)
    attr = _edge_attr(dist2, centers, params["W_rbf"], params["b_rbf"])

    x, y = _embed(x_raw, params["embeddings"], params["W_msg"][0])
    for l in range(N_LAYER):
        agg2 = _sc_scatter(y, attr, ep)
        wm_next = params["W_msg"][l + 1] if l + 1 < N_LAYER else None
        x, y = _layer_update(x, agg2[:, :N_NODES, :], params["W_self"][l],
                             params["W_agg"][l], wm_next)

    return _pool_head(x, batch, unimol_embeddings, params)
